# baseline (speedup 1.0000x reference)
import functools

import jax
import jax.numpy as jnp
from jax import lax
from jax.experimental import pallas as pl
from jax.experimental.pallas import tpu as pltpu
from jax.experimental.pallas import tpu_sc as plsc

N = 10000
E = 320000
D_IN = 128
DH = 16

NC = 2
NS = 16
NW = NC * NS
CH = 128
J = 80
KB = 4
TOT_B = NW * J
J0 = 80
J1 = 2 * J - J0
JMX = max(J0, J1)
E_PAD = TOT_B * CH
N_PAD = 10112
RPT = N_PAD // NS


def _deg_body(dst_hbm, ones_hbm, zeros_hbm, out_hbm, dst_v, ones_v, z_v, acc):
    c = lax.axis_index("c")
    s = lax.axis_index("s")
    wid = s * NC + c

    pltpu.sync_copy(zeros_hbm, z_v)
    pltpu.sync_copy(z_v, acc.at[pl.ds(s * RPT, RPT)])
    pltpu.sync_copy(ones_hbm, ones_v)
    pltpu.sync_copy(dst_hbm.at[pl.ds(wid * J, J)], dst_v)
    plsc.subcore_barrier()

    def step(j, _):
        pltpu.sync_copy(ones_v, acc.at[dst_v.at[j]], add=True)
        return ()

    lax.fori_loop(0, J, step, ())
    plsc.subcore_barrier()
    pltpu.sync_copy(acc.at[pl.ds(s * RPT, RPT)], z_v)
    pltpu.sync_copy(z_v, out_hbm.at[pl.ds(c * N_PAD + s * RPT, RPT)])


def _agg_body(h_hbm, src_hbm, dst_hbm, zrows_hbm, out_hbm,
              src_v, dst_v, rows_v, z_v, gsem, ssemA, ssemB, acc):
    c = lax.axis_index("c")
    s = lax.axis_index("s")
    wid = s * NC + c

    start = jnp.where(c == 0, s * J0, NS * J0 + s * J1)
    n_my = jnp.where(c == 0, J0, J1)

    pltpu.sync_copy(zrows_hbm, z_v)
    pltpu.sync_copy(z_v, acc.at[pl.ds(s * RPT, RPT)])
    pltpu.sync_copy(src_hbm.at[pl.ds(start, JMX)], src_v)
    pltpu.sync_copy(dst_hbm.at[pl.ds(start, JMX)], dst_v)
    plsc.subcore_barrier()

    def drain(set_idx, sem, base):
        for b in range(KB):
            pltpu.make_async_copy(
                rows_v.at[set_idx, b], acc.at[dst_v.at[base + b]], sem
            ).wait()

    def half(set_idx, sem, base):
        g = [pltpu.async_copy(h_hbm.at[src_v.at[base + b]],
                              rows_v.at[set_idx, b], gsem)
             for b in range(KB)]
        for b in range(KB):
            g[b].wait()
        for b in range(KB):
            pltpu.async_copy(rows_v.at[set_idx, b],
                             acc.at[dst_v.at[base + b]], sem, add=True)

    def step(k, _):
        base = k * 2 * KB

        @pl.when(k > 0)
        def _():
            drain(0, ssemA, base - 2 * KB)

        half(0, ssemA, base)

        @pl.when(k > 0)
        def _():
            drain(1, ssemB, base - KB)

        half(1, ssemB, base + KB)
        return ()

    lax.fori_loop(0, n_my // (2 * KB), step, ())
    drain(0, ssemA, n_my - 2 * KB)
    drain(1, ssemB, n_my - KB)
    plsc.subcore_barrier()
    pltpu.sync_copy(acc.at[pl.ds(s * RPT, RPT)], z_v)
    pltpu.sync_copy(z_v, out_hbm.at[pl.ds(c * N_PAD + s * RPT, RPT)])


@functools.cache
def _sc_kernels():
    mesh = plsc.VectorSubcoreMesh(core_axis_name="c", subcore_axis_name="s")
    params = pltpu.CompilerParams(use_tc_tiling_on_sc=False)
    deg = pl.kernel(
        _deg_body,
        mesh=mesh,
        compiler_params=params,
        out_type=jax.ShapeDtypeStruct((NC * N_PAD,), jnp.float32),
        scratch_types=[
            pltpu.VMEM((J, CH), jnp.int32),
            pltpu.VMEM((CH,), jnp.float32),
            pltpu.VMEM((RPT,), jnp.float32),
            pltpu.VMEM_SHARED((N_PAD,), jnp.float32),
        ],
    )
    agg = pl.kernel(
        _agg_body,
        mesh=mesh,
        compiler_params=params,
        out_type=jax.ShapeDtypeStruct((NC * N_PAD, DH), jnp.float32),
        scratch_types=[
            pltpu.VMEM((JMX, CH), jnp.int32),
            pltpu.VMEM((JMX, CH), jnp.int32),
            pltpu.VMEM((2, KB, CH, DH), jnp.float32),
            pltpu.VMEM((RPT, DH), jnp.float32),
            pltpu.SemaphoreType.DMA,
            pltpu.SemaphoreType.DMA,
            pltpu.SemaphoreType.DMA,
            pltpu.VMEM_SHARED((N_PAD, DH), jnp.float32),
        ],
    )
    return deg, agg


NR = N // 8
NR_PAD = N_PAD // 8

def _dinv8(degp_ref):
    deg = degp_ref[0] + degp_ref[1] + 1.0
    dinvm = lax.rsqrt(deg)
    g = lax.broadcasted_iota(jnp.int32, (8, 128), 1) // DH
    r = lax.broadcasted_iota(jnp.int32, (8, 128), 0)
    G = (g == r).astype(jnp.float32)
    return jnp.dot(dinvm, G, preferred_element_type=jnp.float32)


def _in_body(degp_ref, x8_ref, w8_ref, h_ref):
    dinv8 = _dinv8(degp_ref)
    h8 = jnp.dot(x8_ref[...], w8_ref[...], preferred_element_type=jnp.float32)
    h_ref[0:NR, :] = h8 * dinv8[0:NR, :]
    h_ref[NR:NR_PAD, :] = jnp.zeros((NR_PAD - NR, 128), jnp.float32)


def _mid_body(degp_ref, aggp_ref, h1_ref, b_ref, w28_ref, h_ref):
    dinv8 = _dinv8(degp_ref)
    su = aggp_ref[0, 0:NR, :] + aggp_ref[1, 0:NR, :] + h1_ref[0:NR, :]
    z = jnp.maximum(su * dinv8[0:NR, :] + b_ref[...], 0.0)
    h2 = jnp.dot(z, w28_ref[...], preferred_element_type=jnp.float32)
    h_ref[0:NR, :] = h2 * dinv8[0:NR, :]
    h_ref[NR:NR_PAD, :] = jnp.zeros((NR_PAD - NR, 128), jnp.float32)


def _out_body(degp_ref, aggp_ref, h2_ref, b_ref, out_ref):
    dinv8 = _dinv8(degp_ref)
    su = aggp_ref[0, 0:NR, :] + aggp_ref[1, 0:NR, :] + h2_ref[0:NR, :]
    t = su * dinv8[0:NR, :] + b_ref[...]
    gj = lax.broadcasted_iota(jnp.int32, (128, 128), 0) // DH
    gk = lax.broadcasted_iota(jnp.int32, (128, 128), 1) // DH
    GT = (gj == gk).astype(jnp.float32)
    mu = jnp.dot(t, GT / DH, preferred_element_type=jnp.float32)
    u = t - mu
    ssum = jnp.dot(jnp.exp(u), GT, preferred_element_type=jnp.float32)
    out_ref[...] = u - jnp.log(ssum)


_in_call = pl.pallas_call(
    _in_body,
    out_shape=jax.ShapeDtypeStruct((NR_PAD, 128), jnp.float32),
)

_mid_call = pl.pallas_call(
    _mid_body,
    out_shape=jax.ShapeDtypeStruct((NR_PAD, 128), jnp.float32),
)

_out_call = pl.pallas_call(
    _out_body,
    out_shape=jax.ShapeDtypeStruct((NR, 128), jnp.float32),
)


@jax.jit
def kernel(x, edge_index, W1, b1, W2, b2):
    ei = edge_index.astype(jnp.int32)
    pad = jnp.full((E_PAD - E,), N, jnp.int32)
    src = jnp.concatenate([ei[0], pad]).reshape(TOT_B, CH)
    dst = jnp.concatenate([ei[1], pad]).reshape(TOT_B, CH)

    ones_c = jnp.ones((CH,), jnp.float32)
    zeros_r = jnp.zeros((RPT,), jnp.float32)
    zrows = jnp.zeros((RPT, DH), jnp.float32)

    x8 = x.reshape(NR, 8 * D_IN)
    w8 = jnp.kron(jnp.eye(8, dtype=jnp.float32), W1)
    w28 = jnp.kron(jnp.eye(8, dtype=jnp.float32), W2)
    b1t = jnp.tile(b1, 8)[None, :]
    b2t = jnp.tile(b2, 8)[None, :]

    deg_kernel, agg_kernel = _sc_kernels()
    degp8 = deg_kernel(dst, ones_c, zeros_r).reshape(NC, NR_PAD, 8)

    h1p = _in_call(degp8, x8, w8)
    agg1 = agg_kernel(h1p.reshape(N_PAD, DH), src, dst, zrows)
    h2p = _mid_call(degp8, agg1.reshape(NC, NR_PAD, 128), h1p, b1t, w28)
    agg2 = agg_kernel(h2p.reshape(N_PAD, DH), src, dst, zrows)
    out8 = _out_call(degp8, agg2.reshape(NC, NR_PAD, 128), h2p, b2t)
    return out8.reshape(N, DH)


# speedup vs baseline: 1.0721x; 1.0721x over previous
import functools

import jax
import jax.numpy as jnp
from jax import lax
from jax.experimental import pallas as pl
from jax.experimental.pallas import tpu as pltpu
from jax.experimental.pallas import tpu_sc as plsc

N = 10000
E = 320000
D_IN = 128
DH = 16

NC = 2
NS = 16
NW = NC * NS
CH = 128
J = 80
KB = 4
TOT_B = NW * J
J0 = 112
J1 = 2 * J - J0
JMX = max(J0, J1)
E_PAD = TOT_B * CH
N_PAD = 10112
RPT = N_PAD // NS


def _deg_body(dst_hbm, ones_hbm, zeros_hbm, out_hbm, dst_v, ones_v, z_v, acc):
    c = lax.axis_index("c")
    s = lax.axis_index("s")
    wid = s * NC + c

    pltpu.sync_copy(zeros_hbm, z_v)
    pltpu.sync_copy(z_v, acc.at[pl.ds(s * RPT, RPT)])
    pltpu.sync_copy(ones_hbm, ones_v)
    pltpu.sync_copy(dst_hbm.at[pl.ds(wid * J, J)], dst_v)
    plsc.subcore_barrier()

    def step(j, _):
        pltpu.sync_copy(ones_v, acc.at[dst_v.at[j]], add=True)
        return ()

    lax.fori_loop(0, J, step, ())
    plsc.subcore_barrier()
    pltpu.sync_copy(acc.at[pl.ds(s * RPT, RPT)], z_v)
    pltpu.sync_copy(z_v, out_hbm.at[pl.ds(c * N_PAD + s * RPT, RPT)])


def _agg_body(h_hbm, src_hbm, dst_hbm, zrows_hbm, out_hbm,
              src_v, dst_v, rows_v, z_v, gsem, ssemA, ssemB, acc):
    c = lax.axis_index("c")
    s = lax.axis_index("s")
    wid = s * NC + c

    start = jnp.where(c == 0, s * J0, NS * J0 + s * J1)
    n_my = jnp.where(c == 0, J0, J1)

    pltpu.sync_copy(zrows_hbm, z_v)
    pltpu.sync_copy(z_v, acc.at[pl.ds(s * RPT, RPT)])
    pltpu.sync_copy(src_hbm.at[pl.ds(start, JMX)], src_v)
    pltpu.sync_copy(dst_hbm.at[pl.ds(start, JMX)], dst_v)
    plsc.subcore_barrier()

    def drain(set_idx, sem, base):
        for b in range(KB):
            pltpu.make_async_copy(
                rows_v.at[set_idx, b], acc.at[dst_v.at[base + b]], sem
            ).wait()

    def half(set_idx, sem, base):
        g = [pltpu.async_copy(h_hbm.at[src_v.at[base + b]],
                              rows_v.at[set_idx, b], gsem)
             for b in range(KB)]
        for b in range(KB):
            g[b].wait()
        for b in range(KB):
            pltpu.async_copy(rows_v.at[set_idx, b],
                             acc.at[dst_v.at[base + b]], sem, add=True)

    def step(k, _):
        base = k * 2 * KB

        @pl.when(k > 0)
        def _():
            drain(0, ssemA, base - 2 * KB)

        half(0, ssemA, base)

        @pl.when(k > 0)
        def _():
            drain(1, ssemB, base - KB)

        half(1, ssemB, base + KB)
        return ()

    lax.fori_loop(0, n_my // (2 * KB), step, ())
    drain(0, ssemA, n_my - 2 * KB)
    drain(1, ssemB, n_my - KB)
    plsc.subcore_barrier()
    pltpu.sync_copy(acc.at[pl.ds(s * RPT, RPT)], z_v)
    pltpu.sync_copy(z_v, out_hbm.at[pl.ds(c * N_PAD + s * RPT, RPT)])


@functools.cache
def _sc_kernels():
    mesh = plsc.VectorSubcoreMesh(core_axis_name="c", subcore_axis_name="s")
    params = pltpu.CompilerParams(use_tc_tiling_on_sc=False)
    deg = pl.kernel(
        _deg_body,
        mesh=mesh,
        compiler_params=params,
        out_type=jax.ShapeDtypeStruct((NC * N_PAD,), jnp.float32),
        scratch_types=[
            pltpu.VMEM((J, CH), jnp.int32),
            pltpu.VMEM((CH,), jnp.float32),
            pltpu.VMEM((RPT,), jnp.float32),
            pltpu.VMEM_SHARED((N_PAD,), jnp.float32),
        ],
    )
    agg = pl.kernel(
        _agg_body,
        mesh=mesh,
        compiler_params=params,
        out_type=jax.ShapeDtypeStruct((NC * N_PAD, DH), jnp.float32),
        scratch_types=[
            pltpu.VMEM((JMX, CH), jnp.int32),
            pltpu.VMEM((JMX, CH), jnp.int32),
            pltpu.VMEM((2, KB, CH, DH), jnp.float32),
            pltpu.VMEM((RPT, DH), jnp.float32),
            pltpu.SemaphoreType.DMA,
            pltpu.SemaphoreType.DMA,
            pltpu.SemaphoreType.DMA,
            pltpu.VMEM_SHARED((N_PAD, DH), jnp.float32),
        ],
    )
    return deg, agg


NR = N // 8
NR_PAD = N_PAD // 8

def _dinv8(degp_ref):
    deg = degp_ref[0] + degp_ref[1] + 1.0
    dinvm = lax.rsqrt(deg)
    g = lax.broadcasted_iota(jnp.int32, (8, 128), 1) // DH
    r = lax.broadcasted_iota(jnp.int32, (8, 128), 0)
    G = (g == r).astype(jnp.float32)
    return jnp.dot(dinvm, G, preferred_element_type=jnp.float32)


def _in_body(degp_ref, x8_ref, w8_ref, h_ref):
    dinv8 = _dinv8(degp_ref)
    h8 = jnp.dot(x8_ref[...], w8_ref[...], preferred_element_type=jnp.float32)
    h_ref[0:NR, :] = h8 * dinv8[0:NR, :]
    h_ref[NR:NR_PAD, :] = jnp.zeros((NR_PAD - NR, 128), jnp.float32)


def _mid_body(degp_ref, aggp_ref, h1_ref, b_ref, w28_ref, h_ref):
    dinv8 = _dinv8(degp_ref)
    su = aggp_ref[0, 0:NR, :] + aggp_ref[1, 0:NR, :] + h1_ref[0:NR, :]
    z = jnp.maximum(su * dinv8[0:NR, :] + b_ref[...], 0.0)
    h2 = jnp.dot(z, w28_ref[...], preferred_element_type=jnp.float32)
    h_ref[0:NR, :] = h2 * dinv8[0:NR, :]
    h_ref[NR:NR_PAD, :] = jnp.zeros((NR_PAD - NR, 128), jnp.float32)


def _out_body(degp_ref, aggp_ref, h2_ref, b_ref, out_ref):
    dinv8 = _dinv8(degp_ref)
    su = aggp_ref[0, 0:NR, :] + aggp_ref[1, 0:NR, :] + h2_ref[0:NR, :]
    t = su * dinv8[0:NR, :] + b_ref[...]
    gj = lax.broadcasted_iota(jnp.int32, (128, 128), 0) // DH
    gk = lax.broadcasted_iota(jnp.int32, (128, 128), 1) // DH
    GT = (gj == gk).astype(jnp.float32)
    mu = jnp.dot(t, GT / DH, preferred_element_type=jnp.float32)
    u = t - mu
    ssum = jnp.dot(jnp.exp(u), GT, preferred_element_type=jnp.float32)
    out_ref[...] = u - jnp.log(ssum)


_in_call = pl.pallas_call(
    _in_body,
    out_shape=jax.ShapeDtypeStruct((NR_PAD, 128), jnp.float32),
)

_mid_call = pl.pallas_call(
    _mid_body,
    out_shape=jax.ShapeDtypeStruct((NR_PAD, 128), jnp.float32),
)

_out_call = pl.pallas_call(
    _out_body,
    out_shape=jax.ShapeDtypeStruct((NR, 128), jnp.float32),
)


@jax.jit
def kernel(x, edge_index, W1, b1, W2, b2):
    ei = edge_index.astype(jnp.int32)
    pad = jnp.full((E_PAD - E,), N, jnp.int32)
    src = jnp.concatenate([ei[0], pad]).reshape(TOT_B, CH)
    dst = jnp.concatenate([ei[1], pad]).reshape(TOT_B, CH)

    ones_c = jnp.ones((CH,), jnp.float32)
    zeros_r = jnp.zeros((RPT,), jnp.float32)
    zrows = jnp.zeros((RPT, DH), jnp.float32)

    x8 = x.reshape(NR, 8 * D_IN)
    w8 = jnp.kron(jnp.eye(8, dtype=jnp.float32), W1)
    w28 = jnp.kron(jnp.eye(8, dtype=jnp.float32), W2)
    b1t = jnp.tile(b1, 8)[None, :]
    b2t = jnp.tile(b2, 8)[None, :]

    deg_kernel, agg_kernel = _sc_kernels()
    degp8 = deg_kernel(dst, ones_c, zeros_r).reshape(NC, NR_PAD, 8)

    h1p = _in_call(degp8, x8, w8)
    agg1 = agg_kernel(h1p.reshape(N_PAD, DH), src, dst, zrows)
    h2p = _mid_call(degp8, agg1.reshape(NC, NR_PAD, 128), h1p, b1t, w28)
    agg2 = agg_kernel(h2p.reshape(N_PAD, DH), src, dst, zrows)
    out8 = _out_call(degp8, agg2.reshape(NC, NR_PAD, 128), h2p, b2t)
    return out8.reshape(N, DH)


# speedup vs baseline: 1.0869x; 1.0137x over previous
import functools

import jax
import jax.numpy as jnp
from jax import lax
from jax.experimental import pallas as pl
from jax.experimental.pallas import tpu as pltpu
from jax.experimental.pallas import tpu_sc as plsc

N = 10000
E = 320000
D_IN = 128
DH = 16

NC = 2
NS = 16
NW = NC * NS
CH = 128
J = 80
KB = 4
TOT_B = NW * J
J0 = 120
J1 = 2 * J - J0
JMX = max(J0, J1)
E_PAD = TOT_B * CH
N_PAD = 10112
RPT = N_PAD // NS


def _deg_body(dst_hbm, ones_hbm, zeros_hbm, out_hbm, dst_v, ones_v, z_v, acc):
    c = lax.axis_index("c")
    s = lax.axis_index("s")
    wid = s * NC + c

    pltpu.sync_copy(zeros_hbm, z_v)
    pltpu.sync_copy(z_v, acc.at[pl.ds(s * RPT, RPT)])
    pltpu.sync_copy(ones_hbm, ones_v)
    pltpu.sync_copy(dst_hbm.at[pl.ds(wid * J, J)], dst_v)
    plsc.subcore_barrier()

    def step(j, _):
        pltpu.sync_copy(ones_v, acc.at[dst_v.at[j]], add=True)
        return ()

    lax.fori_loop(0, J, step, ())
    plsc.subcore_barrier()
    pltpu.sync_copy(acc.at[pl.ds(s * RPT, RPT)], z_v)
    pltpu.sync_copy(z_v, out_hbm.at[pl.ds(c * N_PAD + s * RPT, RPT)])


def _agg_body(h_hbm, src_hbm, dst_hbm, zrows_hbm, out_hbm,
              src_v, dst_v, rows_v, z_v, gsem, ssemA, ssemB, acc):
    c = lax.axis_index("c")
    s = lax.axis_index("s")
    wid = s * NC + c

    start = jnp.where(c == 0, s * J0, NS * J0 + s * J1)
    n_my = jnp.where(c == 0, J0, J1)

    pltpu.sync_copy(zrows_hbm, z_v)
    pltpu.sync_copy(z_v, acc.at[pl.ds(s * RPT, RPT)])
    pltpu.sync_copy(src_hbm.at[pl.ds(start, JMX)], src_v)
    pltpu.sync_copy(dst_hbm.at[pl.ds(start, JMX)], dst_v)
    plsc.subcore_barrier()

    def drain(set_idx, sem, base):
        for b in range(KB):
            pltpu.make_async_copy(
                rows_v.at[set_idx, b], acc.at[dst_v.at[base + b]], sem
            ).wait()

    def half(set_idx, sem, base):
        g = [pltpu.async_copy(h_hbm.at[src_v.at[base + b]],
                              rows_v.at[set_idx, b], gsem)
             for b in range(KB)]
        for b in range(KB):
            g[b].wait()
        for b in range(KB):
            pltpu.async_copy(rows_v.at[set_idx, b],
                             acc.at[dst_v.at[base + b]], sem, add=True)

    def step(k, _):
        base = k * 2 * KB

        @pl.when(k > 0)
        def _():
            drain(0, ssemA, base - 2 * KB)

        half(0, ssemA, base)

        @pl.when(k > 0)
        def _():
            drain(1, ssemB, base - KB)

        half(1, ssemB, base + KB)
        return ()

    lax.fori_loop(0, n_my // (2 * KB), step, ())
    drain(0, ssemA, n_my - 2 * KB)
    drain(1, ssemB, n_my - KB)
    plsc.subcore_barrier()
    pltpu.sync_copy(acc.at[pl.ds(s * RPT, RPT)], z_v)
    pltpu.sync_copy(z_v, out_hbm.at[pl.ds(c * N_PAD + s * RPT, RPT)])


@functools.cache
def _sc_kernels():
    mesh = plsc.VectorSubcoreMesh(core_axis_name="c", subcore_axis_name="s")
    params = pltpu.CompilerParams(use_tc_tiling_on_sc=False)
    deg = pl.kernel(
        _deg_body,
        mesh=mesh,
        compiler_params=params,
        out_type=jax.ShapeDtypeStruct((NC * N_PAD,), jnp.float32),
        scratch_types=[
            pltpu.VMEM((J, CH), jnp.int32),
            pltpu.VMEM((CH,), jnp.float32),
            pltpu.VMEM((RPT,), jnp.float32),
            pltpu.VMEM_SHARED((N_PAD,), jnp.float32),
        ],
    )
    agg = pl.kernel(
        _agg_body,
        mesh=mesh,
        compiler_params=params,
        out_type=jax.ShapeDtypeStruct((NC * N_PAD, DH), jnp.float32),
        scratch_types=[
            pltpu.VMEM((JMX, CH), jnp.int32),
            pltpu.VMEM((JMX, CH), jnp.int32),
            pltpu.VMEM((2, KB, CH, DH), jnp.float32),
            pltpu.VMEM((RPT, DH), jnp.float32),
            pltpu.SemaphoreType.DMA,
            pltpu.SemaphoreType.DMA,
            pltpu.SemaphoreType.DMA,
            pltpu.VMEM_SHARED((N_PAD, DH), jnp.float32),
        ],
    )
    return deg, agg


NR = N // 8
NR_PAD = N_PAD // 8

def _dinv8(degp_ref):
    deg = degp_ref[0] + degp_ref[1] + 1.0
    dinvm = lax.rsqrt(deg)
    g = lax.broadcasted_iota(jnp.int32, (8, 128), 1) // DH
    r = lax.broadcasted_iota(jnp.int32, (8, 128), 0)
    G = (g == r).astype(jnp.float32)
    return jnp.dot(dinvm, G, preferred_element_type=jnp.float32)


def _in_body(degp_ref, x8_ref, w8_ref, h_ref):
    dinv8 = _dinv8(degp_ref)
    h8 = jnp.dot(x8_ref[...], w8_ref[...], preferred_element_type=jnp.float32)
    h_ref[0:NR, :] = h8 * dinv8[0:NR, :]
    h_ref[NR:NR_PAD, :] = jnp.zeros((NR_PAD - NR, 128), jnp.float32)


def _mid_body(degp_ref, aggp_ref, h1_ref, b_ref, w28_ref, h_ref):
    dinv8 = _dinv8(degp_ref)
    su = aggp_ref[0, 0:NR, :] + aggp_ref[1, 0:NR, :] + h1_ref[0:NR, :]
    z = jnp.maximum(su * dinv8[0:NR, :] + b_ref[...], 0.0)
    h2 = jnp.dot(z, w28_ref[...], preferred_element_type=jnp.float32)
    h_ref[0:NR, :] = h2 * dinv8[0:NR, :]
    h_ref[NR:NR_PAD, :] = jnp.zeros((NR_PAD - NR, 128), jnp.float32)


def _out_body(degp_ref, aggp_ref, h2_ref, b_ref, out_ref):
    dinv8 = _dinv8(degp_ref)
    su = aggp_ref[0, 0:NR, :] + aggp_ref[1, 0:NR, :] + h2_ref[0:NR, :]
    t = su * dinv8[0:NR, :] + b_ref[...]
    gj = lax.broadcasted_iota(jnp.int32, (128, 128), 0) // DH
    gk = lax.broadcasted_iota(jnp.int32, (128, 128), 1) // DH
    GT = (gj == gk).astype(jnp.float32)
    mu = jnp.dot(t, GT / DH, preferred_element_type=jnp.float32)
    u = t - mu
    ssum = jnp.dot(jnp.exp(u), GT, preferred_element_type=jnp.float32)
    out_ref[...] = u - jnp.log(ssum)


_in_call = pl.pallas_call(
    _in_body,
    out_shape=jax.ShapeDtypeStruct((NR_PAD, 128), jnp.float32),
)

_mid_call = pl.pallas_call(
    _mid_body,
    out_shape=jax.ShapeDtypeStruct((NR_PAD, 128), jnp.float32),
)

_out_call = pl.pallas_call(
    _out_body,
    out_shape=jax.ShapeDtypeStruct((NR, 128), jnp.float32),
)


@jax.jit
def kernel(x, edge_index, W1, b1, W2, b2):
    ei = edge_index.astype(jnp.int32)
    pad = jnp.full((E_PAD - E,), N, jnp.int32)
    src = jnp.concatenate([ei[0], pad]).reshape(TOT_B, CH)
    dst = jnp.concatenate([ei[1], pad]).reshape(TOT_B, CH)

    ones_c = jnp.ones((CH,), jnp.float32)
    zeros_r = jnp.zeros((RPT,), jnp.float32)
    zrows = jnp.zeros((RPT, DH), jnp.float32)

    x8 = x.reshape(NR, 8 * D_IN)
    w8 = jnp.kron(jnp.eye(8, dtype=jnp.float32), W1)
    w28 = jnp.kron(jnp.eye(8, dtype=jnp.float32), W2)
    b1t = jnp.tile(b1, 8)[None, :]
    b2t = jnp.tile(b2, 8)[None, :]

    deg_kernel, agg_kernel = _sc_kernels()
    degp8 = deg_kernel(dst, ones_c, zeros_r).reshape(NC, NR_PAD, 8)

    h1p = _in_call(degp8, x8, w8)
    agg1 = agg_kernel(h1p.reshape(N_PAD, DH), src, dst, zrows)
    h2p = _mid_call(degp8, agg1.reshape(NC, NR_PAD, 128), h1p, b1t, w28)
    agg2 = agg_kernel(h2p.reshape(N_PAD, DH), src, dst, zrows)
    out8 = _out_call(degp8, agg2.reshape(NC, NR_PAD, 128), h2p, b2t)
    return out8.reshape(N, DH)


# speedup vs baseline: 1.1178x; 1.0284x over previous
import functools

import jax
import jax.numpy as jnp
from jax import lax
from jax.experimental import pallas as pl
from jax.experimental.pallas import tpu as pltpu
from jax.experimental.pallas import tpu_sc as plsc

N = 10000
E = 320000
D_IN = 128
DH = 16

NC = 2
NS = 16
NW = NC * NS
CH = 128
J = 80
KB = 4
TOT_B = NW * J
J0 = 128
J1 = 2 * J - J0
JMX = max(J0, J1)
E_PAD = TOT_B * CH
N_PAD = 10112
RPT = N_PAD // NS


def _deg_body(dst_hbm, ones_hbm, zeros_hbm, out_hbm, dst_v, ones_v, z_v, acc):
    c = lax.axis_index("c")
    s = lax.axis_index("s")
    wid = s * NC + c

    pltpu.sync_copy(zeros_hbm, z_v)
    pltpu.sync_copy(z_v, acc.at[pl.ds(s * RPT, RPT)])
    pltpu.sync_copy(ones_hbm, ones_v)
    pltpu.sync_copy(dst_hbm.at[pl.ds(wid * J, J)], dst_v)
    plsc.subcore_barrier()

    def step(j, _):
        pltpu.sync_copy(ones_v, acc.at[dst_v.at[j]], add=True)
        return ()

    lax.fori_loop(0, J, step, ())
    plsc.subcore_barrier()
    pltpu.sync_copy(acc.at[pl.ds(s * RPT, RPT)], z_v)
    pltpu.sync_copy(z_v, out_hbm.at[pl.ds(c * N_PAD + s * RPT, RPT)])


def _agg_body(h_hbm, src_hbm, dst_hbm, zrows_hbm, out_hbm,
              src_v, dst_v, rows_v, z_v, gsem, ssemA, ssemB, acc):
    c = lax.axis_index("c")
    s = lax.axis_index("s")
    wid = s * NC + c

    start = jnp.where(c == 0, s * J0, NS * J0 + s * J1)
    n_my = jnp.where(c == 0, J0, J1)

    pltpu.sync_copy(zrows_hbm, z_v)
    pltpu.sync_copy(z_v, acc.at[pl.ds(s * RPT, RPT)])
    pltpu.sync_copy(src_hbm.at[pl.ds(start, JMX)], src_v)
    pltpu.sync_copy(dst_hbm.at[pl.ds(start, JMX)], dst_v)
    plsc.subcore_barrier()

    def drain(set_idx, sem, base):
        for b in range(KB):
            pltpu.make_async_copy(
                rows_v.at[set_idx, b], acc.at[dst_v.at[base + b]], sem
            ).wait()

    def half(set_idx, sem, base):
        g = [pltpu.async_copy(h_hbm.at[src_v.at[base + b]],
                              rows_v.at[set_idx, b], gsem)
             for b in range(KB)]
        for b in range(KB):
            g[b].wait()
        for b in range(KB):
            pltpu.async_copy(rows_v.at[set_idx, b],
                             acc.at[dst_v.at[base + b]], sem, add=True)

    def step(k, _):
        base = k * 2 * KB

        @pl.when(k > 0)
        def _():
            drain(0, ssemA, base - 2 * KB)

        half(0, ssemA, base)

        @pl.when(k > 0)
        def _():
            drain(1, ssemB, base - KB)

        half(1, ssemB, base + KB)
        return ()

    lax.fori_loop(0, n_my // (2 * KB), step, ())
    drain(0, ssemA, n_my - 2 * KB)
    drain(1, ssemB, n_my - KB)
    plsc.subcore_barrier()
    pltpu.sync_copy(acc.at[pl.ds(s * RPT, RPT)], z_v)
    pltpu.sync_copy(z_v, out_hbm.at[pl.ds(c * N_PAD + s * RPT, RPT)])


@functools.cache
def _sc_kernels():
    mesh = plsc.VectorSubcoreMesh(core_axis_name="c", subcore_axis_name="s")
    params = pltpu.CompilerParams(use_tc_tiling_on_sc=False)
    deg = pl.kernel(
        _deg_body,
        mesh=mesh,
        compiler_params=params,
        out_type=jax.ShapeDtypeStruct((NC * N_PAD,), jnp.float32),
        scratch_types=[
            pltpu.VMEM((J, CH), jnp.int32),
            pltpu.VMEM((CH,), jnp.float32),
            pltpu.VMEM((RPT,), jnp.float32),
            pltpu.VMEM_SHARED((N_PAD,), jnp.float32),
        ],
    )
    agg = pl.kernel(
        _agg_body,
        mesh=mesh,
        compiler_params=params,
        out_type=jax.ShapeDtypeStruct((NC * N_PAD, DH), jnp.float32),
        scratch_types=[
            pltpu.VMEM((JMX, CH), jnp.int32),
            pltpu.VMEM((JMX, CH), jnp.int32),
            pltpu.VMEM((2, KB, CH, DH), jnp.float32),
            pltpu.VMEM((RPT, DH), jnp.float32),
            pltpu.SemaphoreType.DMA,
            pltpu.SemaphoreType.DMA,
            pltpu.SemaphoreType.DMA,
            pltpu.VMEM_SHARED((N_PAD, DH), jnp.float32),
        ],
    )
    return deg, agg


NR = N // 8
NR_PAD = N_PAD // 8

def _dinv8(degp_ref):
    deg = degp_ref[0] + degp_ref[1] + 1.0
    dinvm = lax.rsqrt(deg)
    g = lax.broadcasted_iota(jnp.int32, (8, 128), 1) // DH
    r = lax.broadcasted_iota(jnp.int32, (8, 128), 0)
    G = (g == r).astype(jnp.float32)
    return jnp.dot(dinvm, G, preferred_element_type=jnp.float32)


def _in_body(degp_ref, x8_ref, w8_ref, h_ref):
    dinv8 = _dinv8(degp_ref)
    h8 = jnp.dot(x8_ref[...], w8_ref[...], preferred_element_type=jnp.float32)
    h_ref[0:NR, :] = h8 * dinv8[0:NR, :]
    h_ref[NR:NR_PAD, :] = jnp.zeros((NR_PAD - NR, 128), jnp.float32)


def _mid_body(degp_ref, aggp_ref, h1_ref, b_ref, w28_ref, h_ref):
    dinv8 = _dinv8(degp_ref)
    su = aggp_ref[0, 0:NR, :] + aggp_ref[1, 0:NR, :] + h1_ref[0:NR, :]
    z = jnp.maximum(su * dinv8[0:NR, :] + b_ref[...], 0.0)
    h2 = jnp.dot(z, w28_ref[...], preferred_element_type=jnp.float32)
    h_ref[0:NR, :] = h2 * dinv8[0:NR, :]
    h_ref[NR:NR_PAD, :] = jnp.zeros((NR_PAD - NR, 128), jnp.float32)


def _out_body(degp_ref, aggp_ref, h2_ref, b_ref, out_ref):
    dinv8 = _dinv8(degp_ref)
    su = aggp_ref[0, 0:NR, :] + aggp_ref[1, 0:NR, :] + h2_ref[0:NR, :]
    t = su * dinv8[0:NR, :] + b_ref[...]
    gj = lax.broadcasted_iota(jnp.int32, (128, 128), 0) // DH
    gk = lax.broadcasted_iota(jnp.int32, (128, 128), 1) // DH
    GT = (gj == gk).astype(jnp.float32)
    mu = jnp.dot(t, GT / DH, preferred_element_type=jnp.float32)
    u = t - mu
    ssum = jnp.dot(jnp.exp(u), GT, preferred_element_type=jnp.float32)
    out_ref[...] = u - jnp.log(ssum)


_in_call = pl.pallas_call(
    _in_body,
    out_shape=jax.ShapeDtypeStruct((NR_PAD, 128), jnp.float32),
)

_mid_call = pl.pallas_call(
    _mid_body,
    out_shape=jax.ShapeDtypeStruct((NR_PAD, 128), jnp.float32),
)

_out_call = pl.pallas_call(
    _out_body,
    out_shape=jax.ShapeDtypeStruct((NR, 128), jnp.float32),
)


@jax.jit
def kernel(x, edge_index, W1, b1, W2, b2):
    ei = edge_index.astype(jnp.int32)
    pad = jnp.full((E_PAD - E,), N, jnp.int32)
    src = jnp.concatenate([ei[0], pad]).reshape(TOT_B, CH)
    dst = jnp.concatenate([ei[1], pad]).reshape(TOT_B, CH)

    ones_c = jnp.ones((CH,), jnp.float32)
    zeros_r = jnp.zeros((RPT,), jnp.float32)
    zrows = jnp.zeros((RPT, DH), jnp.float32)

    x8 = x.reshape(NR, 8 * D_IN)
    w8 = jnp.kron(jnp.eye(8, dtype=jnp.float32), W1)
    w28 = jnp.kron(jnp.eye(8, dtype=jnp.float32), W2)
    b1t = jnp.tile(b1, 8)[None, :]
    b2t = jnp.tile(b2, 8)[None, :]

    deg_kernel, agg_kernel = _sc_kernels()
    degp8 = deg_kernel(dst, ones_c, zeros_r).reshape(NC, NR_PAD, 8)

    h1p = _in_call(degp8, x8, w8)
    agg1 = agg_kernel(h1p.reshape(N_PAD, DH), src, dst, zrows)
    h2p = _mid_call(degp8, agg1.reshape(NC, NR_PAD, 128), h1p, b1t, w28)
    agg2 = agg_kernel(h2p.reshape(N_PAD, DH), src, dst, zrows)
    out8 = _out_call(degp8, agg2.reshape(NC, NR_PAD, 128), h2p, b2t)
    return out8.reshape(N, DH)
